# Initial kernel scaffold; baseline (speedup 1.0000x reference)
import functools

import jax
import jax.numpy as jnp
from jax import lax
from jax.experimental import pallas as pl
from jax.experimental.pallas import tpu as pltpu
from jax.experimental.pallas import tpu_sc as plsc

N = 10000
D = 128
E = 320000
NC = 2
NS = 16
NW = NC * NS
EPW = E // NW
CH = 128
NFULL = EPW // CH
REM = EPW - NFULL * CH
RPW = N // NS
CW = 16
ZR = 125

_mesh = plsc.VectorSubcoreMesh(core_axis_name="c", subcore_axis_name="s")


def _make_agg(with_count):
    out_types = [jax.ShapeDtypeStruct((NC, N, D), jnp.float32)]
    scratch = [
        pltpu.VMEM_SHARED((N, D), jnp.float32),
        pltpu.VMEM((CH,), jnp.int32),
        pltpu.VMEM((CH,), jnp.int32),
        pltpu.VMEM((CH, D), jnp.float32),
        pltpu.VMEM((REM,), jnp.int32),
        pltpu.VMEM((REM,), jnp.int32),
        pltpu.VMEM((REM, D), jnp.float32),
        pltpu.VMEM((ZR, D), jnp.float32),
    ]
    if with_count:
        out_types.append(jax.ShapeDtypeStruct((NC, N, CW), jnp.float32))
        scratch += [
            pltpu.VMEM_SHARED((N, CW), jnp.float32),
            pltpu.VMEM((CH, CW), jnp.float32),
            pltpu.VMEM((REM, CW), jnp.float32),
            pltpu.VMEM((RPW, CW), jnp.float32),
        ]

    def body(*refs):
        if with_count:
            (x_hbm, src_hbm, dst_hbm, sum_hbm, cnt_hbm,
             acc, srcv, dstv, rows, srcr, dstr, rowsr, zbuf,
             cacc, ones, onesr, zsm) = refs
        else:
            (x_hbm, src_hbm, dst_hbm, sum_hbm,
             acc, srcv, dstv, rows, srcr, dstr, rowsr, zbuf) = refs

        c = lax.axis_index("c")
        s = lax.axis_index("s")
        wid = c * NS + s

        @pl.loop(0, ZR)
        def _(i):
            @pl.loop(0, D, step=16)
            def _(j):
                zbuf[i, pl.ds(j, 16)] = jnp.zeros((16,), jnp.float32)

        @pl.loop(0, RPW // ZR)
        def _(k):
            pltpu.sync_copy(zbuf, acc.at[pl.ds(s * RPW + k * ZR, ZR)])

        if with_count:
            @pl.loop(0, CH)
            def _(i):
                ones[i, pl.ds(0, CW)] = jnp.ones((CW,), jnp.float32)

            @pl.loop(0, REM)
            def _(i):
                onesr[i, pl.ds(0, CW)] = jnp.ones((CW,), jnp.float32)

            @pl.loop(0, RPW)
            def _(i):
                zsm[i, pl.ds(0, CW)] = jnp.zeros((CW,), jnp.float32)

            pltpu.sync_copy(zsm, cacc.at[pl.ds(s * RPW, RPW)])

        plsc.subcore_barrier()

        base = wid * EPW

        @pl.loop(0, NFULL)
        def _(i):
            off = base + i * CH
            pltpu.sync_copy(src_hbm.at[pl.ds(off, CH)], srcv)
            pltpu.sync_copy(dst_hbm.at[pl.ds(off, CH)], dstv)
            pltpu.sync_copy(x_hbm.at[srcv], rows)
            pltpu.sync_copy(rows, acc.at[dstv], add=True)
            if with_count:
                pltpu.sync_copy(ones, cacc.at[dstv], add=True)

        offr = base + NFULL * CH
        pltpu.sync_copy(src_hbm.at[pl.ds(offr, REM)], srcr)
        pltpu.sync_copy(dst_hbm.at[pl.ds(offr, REM)], dstr)
        pltpu.sync_copy(x_hbm.at[srcr], rowsr)
        pltpu.sync_copy(rowsr, acc.at[dstr], add=True)
        if with_count:
            pltpu.sync_copy(onesr, cacc.at[dstr], add=True)

        plsc.subcore_barrier()

        pltpu.sync_copy(acc.at[pl.ds(s * RPW, RPW)],
                        sum_hbm.at[c, pl.ds(s * RPW, RPW)])
        if with_count:
            pltpu.sync_copy(cacc.at[pl.ds(s * RPW, RPW)],
                            cnt_hbm.at[c, pl.ds(s * RPW, RPW)])

    return pl.kernel(
        body,
        out_type=tuple(out_types) if with_count else out_types[0],
        mesh=_mesh,
        scratch_types=scratch,
    )


_agg_cnt = _make_agg(True)
_agg = _make_agg(False)

RB = 1000


def _combine_body(relu, p_ref, c_ref, x_ref, wl_ref, b_ref, wr_ref, o_ref):
    psum = p_ref[0] + p_ref[1]
    cnt = c_ref[0, :, 0:1] + c_ref[1, :, 0:1]
    mean = psum / jnp.maximum(cnt, 1.0)
    h = (jnp.dot(mean, wl_ref[...], preferred_element_type=jnp.float32)
         + b_ref[...]
         + jnp.dot(x_ref[...], wr_ref[...], preferred_element_type=jnp.float32))
    if relu:
        h = jnp.maximum(h, 0.0)
    o_ref[...] = h


def _combine(relu, p, cnt, x, wl_t, b, wr_t):
    return pl.pallas_call(
        functools.partial(_combine_body, relu),
        out_shape=jax.ShapeDtypeStruct((N, D), jnp.float32),
        grid=(N // RB,),
        in_specs=[
            pl.BlockSpec((NC, RB, D), lambda i: (0, i, 0)),
            pl.BlockSpec((NC, RB, CW), lambda i: (0, i, 0)),
            pl.BlockSpec((RB, D), lambda i: (i, 0)),
            pl.BlockSpec((D, D), lambda i: (0, 0)),
            pl.BlockSpec((1, D), lambda i: (0, 0)),
            pl.BlockSpec((D, D), lambda i: (0, 0)),
        ],
        out_specs=pl.BlockSpec((RB, D), lambda i: (i, 0)),
    )(p, cnt, x, wl_t, b, wr_t)


def kernel(x, edge_index, W_l0, b_l0, W_r0, W_l1, b_l1, W_r1, W_l2, b_l2, W_r2):
    src = edge_index[0]
    dst = edge_index[1]

    p, cnt = _agg_cnt(x, src, dst)
    h = _combine(True, p, cnt, x, W_l0.T, b_l0.reshape(1, D), W_r0.T)
    p = _agg(h, src, dst)
    h = _combine(True, p, cnt, h, W_l1.T, b_l1.reshape(1, D), W_r1.T)
    p = _agg(h, src, dst)
    h = _combine(False, p, cnt, h, W_l2.T, b_l2.reshape(1, D), W_r2.T)
    return h


# speedup vs baseline: 4.8222x; 4.8222x over previous
import functools

import jax
import jax.numpy as jnp
from jax import lax
from jax.experimental import pallas as pl
from jax.experimental.pallas import tpu as pltpu
from jax.experimental.pallas import tpu_sc as plsc

N = 10000
D = 128
E = 320000
NC = 2
NS = 16
NW = NC * NS
EPW = E // NW
CH = 80
NCHUNK = EPW // CH
BR = 80
NB = N // BR

_mesh = plsc.VectorSubcoreMesh(core_axis_name="c", subcore_axis_name="s")


def _agg_body(x_hbm, src_hbm, dst_hbm, sum_hbm, acc, rows, srcv, dstv):
    c = lax.axis_index("c")
    s = lax.axis_index("s")
    wid = c * NS + s

    @pl.loop(0, CH)
    def _(i):
        @pl.loop(0, D, step=16)
        def _(j):
            rows[i, pl.ds(j, 16)] = jnp.zeros((16,), jnp.float32)

    @pl.loop(s, NB, step=NS)
    def _(bk):
        pltpu.sync_copy(rows, acc.at[pl.ds(bk * BR, BR)])

    plsc.subcore_barrier()

    base = wid * EPW

    @pl.loop(0, NCHUNK)
    def _(i):
        off = base + i * CH
        pltpu.sync_copy(src_hbm.at[pl.ds(off, CH)], srcv)
        pltpu.sync_copy(dst_hbm.at[pl.ds(off, CH)], dstv)
        pltpu.sync_copy(x_hbm.at[srcv], rows)
        pltpu.sync_copy(rows, acc.at[dstv], add=True)

    plsc.subcore_barrier()

    @pl.loop(s, NB, step=NS)
    def _(bk):
        pltpu.sync_copy(acc.at[pl.ds(bk * BR, BR)], rows)
        pltpu.sync_copy(rows, sum_hbm.at[c, pl.ds(bk * BR, BR)])


_agg = pl.kernel(
    _agg_body,
    out_type=jax.ShapeDtypeStruct((NC, N, D), jnp.float32),
    mesh=_mesh,
    scratch_types=[
        pltpu.VMEM_SHARED((N, D), jnp.float32),
        pltpu.VMEM((CH, D), jnp.float32),
        pltpu.VMEM((CH,), jnp.int32),
        pltpu.VMEM((CH,), jnp.int32),
    ],
)


def _cnt_body(dst_hbm, cnt_hbm, cacc, ones, dstv):
    c = lax.axis_index("c")
    s = lax.axis_index("s")
    wid = c * NS + s

    @pl.loop(0, CH)
    def _(i):
        @pl.loop(0, D, step=16)
        def _(j):
            ones[i, pl.ds(j, 16)] = jnp.zeros((16,), jnp.float32)

    @pl.loop(s, NB, step=NS)
    def _(bk):
        pltpu.sync_copy(ones, cacc.at[pl.ds(bk * BR, BR)])

    @pl.loop(0, CH)
    def _(i):
        @pl.loop(0, D, step=16)
        def _(j):
            ones[i, pl.ds(j, 16)] = jnp.ones((16,), jnp.float32)

    plsc.subcore_barrier()

    base = wid * EPW

    @pl.loop(0, NCHUNK)
    def _(i):
        off = base + i * CH
        pltpu.sync_copy(dst_hbm.at[pl.ds(off, CH)], dstv)
        pltpu.sync_copy(ones, cacc.at[dstv], add=True)

    plsc.subcore_barrier()

    @pl.loop(s, NB, step=NS)
    def _(bk):
        pltpu.sync_copy(cacc.at[pl.ds(bk * BR, BR)], ones)
        pltpu.sync_copy(ones, cnt_hbm.at[c, pl.ds(bk * BR, BR)])


_cnt = pl.kernel(
    _cnt_body,
    out_type=jax.ShapeDtypeStruct((NC, N, D), jnp.float32),
    mesh=_mesh,
    scratch_types=[
        pltpu.VMEM_SHARED((N, D), jnp.float32),
        pltpu.VMEM((CH, D), jnp.float32),
        pltpu.VMEM((CH,), jnp.int32),
    ],
)

RB = 1000


def _combine_body(relu, p_ref, c_ref, x_ref, wl_ref, b_ref, wr_ref, o_ref):
    psum = p_ref[0] + p_ref[1]
    cnt = c_ref[0, :, 0:1] + c_ref[1, :, 0:1]
    mean = psum / jnp.maximum(cnt, 1.0)
    h = (jnp.dot(mean, wl_ref[...], preferred_element_type=jnp.float32)
         + b_ref[...]
         + jnp.dot(x_ref[...], wr_ref[...], preferred_element_type=jnp.float32))
    if relu:
        h = jnp.maximum(h, 0.0)
    o_ref[...] = h


def _combine(relu, p, cnt, x, wl_t, b, wr_t):
    return pl.pallas_call(
        functools.partial(_combine_body, relu),
        out_shape=jax.ShapeDtypeStruct((N, D), jnp.float32),
        grid=(N // RB,),
        in_specs=[
            pl.BlockSpec((NC, RB, D), lambda i: (0, i, 0)),
            pl.BlockSpec((NC, RB, D), lambda i: (0, i, 0)),
            pl.BlockSpec((RB, D), lambda i: (i, 0)),
            pl.BlockSpec((D, D), lambda i: (0, 0)),
            pl.BlockSpec((1, D), lambda i: (0, 0)),
            pl.BlockSpec((D, D), lambda i: (0, 0)),
        ],
        out_specs=pl.BlockSpec((RB, D), lambda i: (i, 0)),
    )(p, cnt, x, wl_t, b, wr_t)


def kernel(x, edge_index, W_l0, b_l0, W_r0, W_l1, b_l1, W_r1, W_l2, b_l2, W_r2):
    src = edge_index[0]
    dst = edge_index[1]

    cnt = _cnt(dst)
    p = _agg(x, src, dst)
    h = _combine(True, p, cnt, x, W_l0.T, b_l0.reshape(1, D), W_r0.T)
    p = _agg(h, src, dst)
    h = _combine(True, p, cnt, h, W_l1.T, b_l1.reshape(1, D), W_r1.T)
    p = _agg(h, src, dst)
    h = _combine(False, p, cnt, h, W_l2.T, b_l2.reshape(1, D), W_r2.T)
    return h


# speedup vs baseline: 8.3270x; 1.7268x over previous
import functools

import jax
import jax.numpy as jnp
from jax import lax
from jax.experimental import pallas as pl
from jax.experimental.pallas import tpu as pltpu
from jax.experimental.pallas import tpu_sc as plsc

N = 10000
D = 128
E = 320000
NC = 2
NS = 16
NW = NC * NS
EPW = E // NW
CH = 80
NCHUNK = EPW // CH
BR = 80
NB = N // BR
CCH = 80
CNCH = EPW // CCH
CNB = N // CCH

_mesh = plsc.VectorSubcoreMesh(core_axis_name="c", subcore_axis_name="s")


def _agg_body(x_hbm, src_hbm, dst_hbm, sum_hbm, acc,
              rows0, rows1, srcbig, dstv0, dstv1,
              g0, g1, s0, s1, d0, d1, isem):
    c = lax.axis_index("c")
    s = lax.axis_index("s")
    wid = c * NS + s
    base = wid * EPW

    pltpu.async_copy(src_hbm.at[pl.ds(base, EPW)], srcbig, isem)

    @pl.loop(0, CH)
    def _(i):
        @pl.loop(0, D, step=16)
        def _(j):
            rows0[i, pl.ds(j, 16)] = jnp.zeros((16,), jnp.float32)

    @pl.loop(s, NB, step=NS)
    def _(bk):
        pltpu.sync_copy(rows0, acc.at[pl.ds(bk * BR, BR)])

    pltpu.make_async_copy(src_hbm.at[pl.ds(base, EPW)], srcbig, isem).wait()

    plsc.subcore_barrier()

    pltpu.async_copy(dst_hbm.at[pl.ds(base, CH)], dstv0, d0)
    pltpu.async_copy(dst_hbm.at[pl.ds(base + CH, CH)], dstv1, d1)
    pltpu.async_copy(x_hbm.at[srcbig.at[pl.ds(0, CH)]], rows0, g0)
    pltpu.async_copy(x_hbm.at[srcbig.at[pl.ds(CH, CH)]], rows1, g1)

    @pl.loop(0, (NCHUNK - 1) // 2)
    def _(k):
        i = 2 * k
        pltpu.make_async_copy(x_hbm.at[srcbig.at[pl.ds(i * CH, CH)]],
                              rows0, g0).wait()
        pltpu.make_async_copy(dst_hbm.at[pl.ds(base + i * CH, CH)],
                              dstv0, d0).wait()
        pltpu.async_copy(rows0, acc.at[dstv0], s0, add=True)

        pltpu.make_async_copy(x_hbm.at[srcbig.at[pl.ds((i + 1) * CH, CH)]],
                              rows1, g1).wait()
        pltpu.make_async_copy(dst_hbm.at[pl.ds(base + (i + 1) * CH, CH)],
                              dstv1, d1).wait()
        pltpu.async_copy(rows1, acc.at[dstv1], s1, add=True)

        pltpu.make_async_copy(rows0, acc.at[dstv0], s0).wait()
        pltpu.async_copy(dst_hbm.at[pl.ds(base + (i + 2) * CH, CH)], dstv0, d0)
        pltpu.async_copy(x_hbm.at[srcbig.at[pl.ds((i + 2) * CH, CH)]], rows0, g0)

        pltpu.make_async_copy(rows1, acc.at[dstv1], s1).wait()

        @pl.when(i + 3 < NCHUNK)
        def _():
            pltpu.async_copy(dst_hbm.at[pl.ds(base + (i + 3) * CH, CH)],
                             dstv1, d1)
            pltpu.async_copy(x_hbm.at[srcbig.at[pl.ds((i + 3) * CH, CH)]],
                             rows1, g1)

    last = NCHUNK - 1
    pltpu.make_async_copy(x_hbm.at[srcbig.at[pl.ds(last * CH, CH)]],
                          rows0, g0).wait()
    pltpu.make_async_copy(dst_hbm.at[pl.ds(base + last * CH, CH)],
                          dstv0, d0).wait()
    pltpu.sync_copy(rows0, acc.at[dstv0], add=True)

    plsc.subcore_barrier()

    @pl.loop(s, NB, step=NS)
    def _(bk):
        pltpu.sync_copy(acc.at[pl.ds(bk * BR, BR)], rows0)
        pltpu.sync_copy(rows0, sum_hbm.at[c, pl.ds(bk * BR, BR)])


_agg = pl.kernel(
    _agg_body,
    out_type=jax.ShapeDtypeStruct((NC, N, D), jnp.float32),
    mesh=_mesh,
    scratch_types=[
        pltpu.VMEM_SHARED((N, D), jnp.float32),
        pltpu.VMEM((CH, D), jnp.float32),
        pltpu.VMEM((CH, D), jnp.float32),
        pltpu.VMEM((EPW,), jnp.int32),
        pltpu.VMEM((CH,), jnp.int32),
        pltpu.VMEM((CH,), jnp.int32),
        pltpu.SemaphoreType.DMA,
        pltpu.SemaphoreType.DMA,
        pltpu.SemaphoreType.DMA,
        pltpu.SemaphoreType.DMA,
        pltpu.SemaphoreType.DMA,
        pltpu.SemaphoreType.DMA,
        pltpu.SemaphoreType.DMA,
    ],
)


def _cnt_body(dst_hbm, cnt_hbm, cacc, ones, dstv):
    c = lax.axis_index("c")
    s = lax.axis_index("s")
    wid = c * NS + s

    @pl.loop(0, CCH)
    def _(i):
        @pl.loop(0, D, step=16)
        def _(j):
            ones[i, pl.ds(j, 16)] = jnp.zeros((16,), jnp.float32)

    @pl.loop(s, CNB, step=NS)
    def _(bk):
        pltpu.sync_copy(ones, cacc.at[pl.ds(bk * CCH, CCH)])

    @pl.loop(0, CCH)
    def _(i):
        @pl.loop(0, D, step=16)
        def _(j):
            ones[i, pl.ds(j, 16)] = jnp.ones((16,), jnp.float32)

    plsc.subcore_barrier()

    base = wid * EPW

    @pl.loop(0, CNCH)
    def _(i):
        off = base + i * CCH
        pltpu.sync_copy(dst_hbm.at[pl.ds(off, CCH)], dstv)
        pltpu.sync_copy(ones, cacc.at[dstv], add=True)

    plsc.subcore_barrier()

    @pl.loop(s, CNB, step=NS)
    def _(bk):
        pltpu.sync_copy(cacc.at[pl.ds(bk * CCH, CCH)], ones)
        pltpu.sync_copy(ones, cnt_hbm.at[c, pl.ds(bk * CCH, CCH)])


_cnt = pl.kernel(
    _cnt_body,
    out_type=jax.ShapeDtypeStruct((NC, N, D), jnp.float32),
    mesh=_mesh,
    scratch_types=[
        pltpu.VMEM_SHARED((N, D), jnp.float32),
        pltpu.VMEM((CCH, D), jnp.float32),
        pltpu.VMEM((CCH,), jnp.int32),
    ],
)

RB = 1000


def _combine_body(relu, p_ref, c_ref, x_ref, wl_ref, b_ref, wr_ref, o_ref):
    psum = p_ref[0] + p_ref[1]
    cnt = c_ref[0, :, 0:1] + c_ref[1, :, 0:1]
    mean = psum / jnp.maximum(cnt, 1.0)
    h = (jnp.dot(mean, wl_ref[...], preferred_element_type=jnp.float32)
         + b_ref[...]
         + jnp.dot(x_ref[...], wr_ref[...], preferred_element_type=jnp.float32))
    if relu:
        h = jnp.maximum(h, 0.0)
    o_ref[...] = h


def _combine(relu, p, cnt, x, wl_t, b, wr_t):
    return pl.pallas_call(
        functools.partial(_combine_body, relu),
        out_shape=jax.ShapeDtypeStruct((N, D), jnp.float32),
        grid=(N // RB,),
        in_specs=[
            pl.BlockSpec((NC, RB, D), lambda i: (0, i, 0)),
            pl.BlockSpec((NC, RB, D), lambda i: (0, i, 0)),
            pl.BlockSpec((RB, D), lambda i: (i, 0)),
            pl.BlockSpec((D, D), lambda i: (0, 0)),
            pl.BlockSpec((1, D), lambda i: (0, 0)),
            pl.BlockSpec((D, D), lambda i: (0, 0)),
        ],
        out_specs=pl.BlockSpec((RB, D), lambda i: (i, 0)),
    )(p, cnt, x, wl_t, b, wr_t)


def kernel(x, edge_index, W_l0, b_l0, W_r0, W_l1, b_l1, W_r1, W_l2, b_l2, W_r2):
    src = edge_index[0]
    dst = edge_index[1]

    cnt = _cnt(dst)
    p = _agg(x, src, dst)
    h = _combine(True, p, cnt, x, W_l0.T, b_l0.reshape(1, D), W_r0.T)
    p = _agg(h, src, dst)
    h = _combine(True, p, cnt, h, W_l1.T, b_l1.reshape(1, D), W_r1.T)
    p = _agg(h, src, dst)
    h = _combine(False, p, cnt, h, W_l2.T, b_l2.reshape(1, D), W_r2.T)
    return h


# speedup vs baseline: 8.7885x; 1.0554x over previous
import functools

import jax
import jax.numpy as jnp
from jax import lax
from jax.experimental import pallas as pl
from jax.experimental.pallas import tpu as pltpu
from jax.experimental.pallas import tpu_sc as plsc

N = 10000
D = 128
E = 320000
NC = 2
NS = 16
NW = NC * NS
EPW = E // NW
CH = 80
NCHUNK = EPW // CH
BR = 80
NB = N // BR
CCH = 80
CNCH = EPW // CCH
CNB = N // CCH

_mesh = plsc.VectorSubcoreMesh(core_axis_name="c", subcore_axis_name="s")


def _agg_body(x_hbm, src_hbm, dst_hbm, sum_hbm, acc,
              rows0, rows1, srcbig, dstv0, dstv1,
              g0, g1, s0, s1, d0, d1, isem):
    c = lax.axis_index("c")
    s = lax.axis_index("s")
    wid = c * NS + s
    base = wid * EPW

    pltpu.async_copy(src_hbm.at[pl.ds(base, EPW)], srcbig, isem)

    @pl.loop(0, CH)
    def _(i):
        @pl.loop(0, D, step=16)
        def _(j):
            rows0[i, pl.ds(j, 16)] = jnp.zeros((16,), jnp.float32)

    @pl.loop(s, NB, step=NS)
    def _(bk):
        pltpu.sync_copy(rows0, acc.at[pl.ds(bk * BR, BR)])

    pltpu.make_async_copy(src_hbm.at[pl.ds(base, EPW)], srcbig, isem).wait()

    plsc.subcore_barrier()

    pltpu.async_copy(dst_hbm.at[pl.ds(base, CH)], dstv0, d0)
    pltpu.async_copy(dst_hbm.at[pl.ds(base + CH, CH)], dstv1, d1)
    pltpu.async_copy(x_hbm.at[srcbig.at[pl.ds(0, CH)]], rows0, g0)
    pltpu.async_copy(x_hbm.at[srcbig.at[pl.ds(CH, CH)]], rows1, g1)

    @pl.loop(0, (NCHUNK - 1) // 2)
    def _(k):
        i = 2 * k
        pltpu.make_async_copy(x_hbm.at[srcbig.at[pl.ds(i * CH, CH)]],
                              rows0, g0).wait()
        pltpu.make_async_copy(dst_hbm.at[pl.ds(base + i * CH, CH)],
                              dstv0, d0).wait()
        pltpu.async_copy(rows0, acc.at[dstv0], s0, add=True)

        pltpu.make_async_copy(x_hbm.at[srcbig.at[pl.ds((i + 1) * CH, CH)]],
                              rows1, g1).wait()
        pltpu.make_async_copy(dst_hbm.at[pl.ds(base + (i + 1) * CH, CH)],
                              dstv1, d1).wait()
        pltpu.async_copy(rows1, acc.at[dstv1], s1, add=True)

        pltpu.make_async_copy(rows0, acc.at[dstv0], s0).wait()
        pltpu.async_copy(dst_hbm.at[pl.ds(base + (i + 2) * CH, CH)], dstv0, d0)
        pltpu.async_copy(x_hbm.at[srcbig.at[pl.ds((i + 2) * CH, CH)]], rows0, g0)

        pltpu.make_async_copy(rows1, acc.at[dstv1], s1).wait()

        @pl.when(i + 3 < NCHUNK)
        def _():
            pltpu.async_copy(dst_hbm.at[pl.ds(base + (i + 3) * CH, CH)],
                             dstv1, d1)
            pltpu.async_copy(x_hbm.at[srcbig.at[pl.ds((i + 3) * CH, CH)]],
                             rows1, g1)

    last = NCHUNK - 1
    pltpu.make_async_copy(x_hbm.at[srcbig.at[pl.ds(last * CH, CH)]],
                          rows0, g0).wait()
    pltpu.make_async_copy(dst_hbm.at[pl.ds(base + last * CH, CH)],
                          dstv0, d0).wait()
    pltpu.sync_copy(rows0, acc.at[dstv0], add=True)

    plsc.subcore_barrier()

    @pl.loop(s, NB, step=NS)
    def _(bk):
        pltpu.sync_copy(acc.at[pl.ds(bk * BR, BR)], rows0)
        pltpu.sync_copy(rows0, sum_hbm.at[c, pl.ds(bk * BR, BR)])


_agg = pl.kernel(
    _agg_body,
    out_type=jax.ShapeDtypeStruct((NC, N, D), jnp.float32),
    mesh=_mesh,
    scratch_types=[
        pltpu.VMEM_SHARED((N, D), jnp.float32),
        pltpu.VMEM((CH, D), jnp.float32),
        pltpu.VMEM((CH, D), jnp.float32),
        pltpu.VMEM((EPW,), jnp.int32),
        pltpu.VMEM((CH,), jnp.int32),
        pltpu.VMEM((CH,), jnp.int32),
        pltpu.SemaphoreType.DMA,
        pltpu.SemaphoreType.DMA,
        pltpu.SemaphoreType.DMA,
        pltpu.SemaphoreType.DMA,
        pltpu.SemaphoreType.DMA,
        pltpu.SemaphoreType.DMA,
        pltpu.SemaphoreType.DMA,
    ],
)


def _cnt_body(dst_hbm, cnt_hbm, cacc, ones, dstv0, dstv1, d0, d1, s0, s1):
    c = lax.axis_index("c")
    s = lax.axis_index("s")
    wid = c * NS + s
    base = wid * EPW

    @pl.loop(0, CCH)
    def _(i):
        @pl.loop(0, D, step=16)
        def _(j):
            ones[i, pl.ds(j, 16)] = jnp.zeros((16,), jnp.float32)

    @pl.loop(s, CNB, step=NS)
    def _(bk):
        pltpu.sync_copy(ones, cacc.at[pl.ds(bk * CCH, CCH)])

    @pl.loop(0, CCH)
    def _(i):
        @pl.loop(0, D, step=16)
        def _(j):
            ones[i, pl.ds(j, 16)] = jnp.ones((16,), jnp.float32)

    plsc.subcore_barrier()

    pltpu.async_copy(dst_hbm.at[pl.ds(base, CCH)], dstv0, d0)
    pltpu.async_copy(dst_hbm.at[pl.ds(base + CCH, CCH)], dstv1, d1)

    @pl.loop(0, (CNCH - 1) // 2)
    def _(k):
        i = 2 * k
        pltpu.make_async_copy(dst_hbm.at[pl.ds(base + i * CCH, CCH)],
                              dstv0, d0).wait()
        pltpu.async_copy(ones, cacc.at[dstv0], s0, add=True)
        pltpu.make_async_copy(dst_hbm.at[pl.ds(base + (i + 1) * CCH, CCH)],
                              dstv1, d1).wait()
        pltpu.async_copy(ones, cacc.at[dstv1], s1, add=True)
        pltpu.make_async_copy(ones, cacc.at[dstv0], s0).wait()
        pltpu.async_copy(dst_hbm.at[pl.ds(base + (i + 2) * CCH, CCH)], dstv0, d0)
        pltpu.make_async_copy(ones, cacc.at[dstv1], s1).wait()

        @pl.when(i + 3 < CNCH)
        def _():
            pltpu.async_copy(dst_hbm.at[pl.ds(base + (i + 3) * CCH, CCH)],
                             dstv1, d1)

    last = CNCH - 1
    pltpu.make_async_copy(dst_hbm.at[pl.ds(base + last * CCH, CCH)],
                          dstv0, d0).wait()
    pltpu.sync_copy(ones, cacc.at[dstv0], add=True)

    plsc.subcore_barrier()

    @pl.loop(s, CNB, step=NS)
    def _(bk):
        pltpu.sync_copy(cacc.at[pl.ds(bk * CCH, CCH)], ones)
        pltpu.sync_copy(ones, cnt_hbm.at[c, pl.ds(bk * CCH, CCH)])


_cnt = pl.kernel(
    _cnt_body,
    out_type=jax.ShapeDtypeStruct((NC, N, D), jnp.float32),
    mesh=_mesh,
    scratch_types=[
        pltpu.VMEM_SHARED((N, D), jnp.float32),
        pltpu.VMEM((CCH, D), jnp.float32),
        pltpu.VMEM((CCH,), jnp.int32),
        pltpu.VMEM((CCH,), jnp.int32),
        pltpu.SemaphoreType.DMA,
        pltpu.SemaphoreType.DMA,
        pltpu.SemaphoreType.DMA,
        pltpu.SemaphoreType.DMA,
    ],
)

RB = 1000


def _combine_body(relu, p_ref, c_ref, x_ref, wl_ref, b_ref, wr_ref, o_ref):
    psum = p_ref[0] + p_ref[1]
    cnt = c_ref[0, :, 0:1] + c_ref[1, :, 0:1]
    mean = psum / jnp.maximum(cnt, 1.0)
    h = (jnp.dot(mean, wl_ref[...], preferred_element_type=jnp.float32)
         + b_ref[...]
         + jnp.dot(x_ref[...], wr_ref[...], preferred_element_type=jnp.float32))
    if relu:
        h = jnp.maximum(h, 0.0)
    o_ref[...] = h


def _combine(relu, p, cnt, x, wl_t, b, wr_t):
    return pl.pallas_call(
        functools.partial(_combine_body, relu),
        out_shape=jax.ShapeDtypeStruct((N, D), jnp.float32),
        grid=(N // RB,),
        in_specs=[
            pl.BlockSpec((NC, RB, D), lambda i: (0, i, 0)),
            pl.BlockSpec((NC, RB, D), lambda i: (0, i, 0)),
            pl.BlockSpec((RB, D), lambda i: (i, 0)),
            pl.BlockSpec((D, D), lambda i: (0, 0)),
            pl.BlockSpec((1, D), lambda i: (0, 0)),
            pl.BlockSpec((D, D), lambda i: (0, 0)),
        ],
        out_specs=pl.BlockSpec((RB, D), lambda i: (i, 0)),
    )(p, cnt, x, wl_t, b, wr_t)


def kernel(x, edge_index, W_l0, b_l0, W_r0, W_l1, b_l1, W_r1, W_l2, b_l2, W_r2):
    src = edge_index[0]
    dst = edge_index[1]

    cnt = _cnt(dst)
    p = _agg(x, src, dst)
    h = _combine(True, p, cnt, x, W_l0.T, b_l0.reshape(1, D), W_r0.T)
    p = _agg(h, src, dst)
    h = _combine(True, p, cnt, h, W_l1.T, b_l1.reshape(1, D), W_r1.T)
    p = _agg(h, src, dst)
    h = _combine(False, p, cnt, h, W_l2.T, b_l2.reshape(1, D), W_r2.T)
    return h


# speedup vs baseline: 10.4644x; 1.1907x over previous
import functools

import jax
import jax.numpy as jnp
from jax import lax
from jax.experimental import pallas as pl
from jax.experimental.pallas import tpu as pltpu
from jax.experimental.pallas import tpu_sc as plsc

N = 10000
D = 128
E = 320000
NC = 2
NS = 16
NW = NC * NS
EPW = E // NW
CH = 80
NCHUNK = EPW // CH
BR = 80
NB = N // BR
CCH = 80
CNCH = EPW // CCH
CNB = N // CCH

_mesh = plsc.VectorSubcoreMesh(core_axis_name="c", subcore_axis_name="s")


def _agg_body(x_hbm, src_hbm, dst_hbm, sum_hbm, acc,
              rows0, rows1, rows2, srcbig, dstv0, dstv1, dstv2,
              g0, g1, g2, s0, s1, s2, d0, d1, d2, isem):
    c = lax.axis_index("c")
    s = lax.axis_index("s")
    wid = c * NS + s
    base = wid * EPW

    pltpu.async_copy(src_hbm.at[pl.ds(base, EPW)], srcbig, isem)

    @pl.loop(0, CH)
    def _(i):
        @pl.loop(0, D, step=16)
        def _(j):
            rows0[i, pl.ds(j, 16)] = jnp.zeros((16,), jnp.float32)

    @pl.loop(s, NB, step=NS)
    def _(bk):
        pltpu.sync_copy(rows0, acc.at[pl.ds(bk * BR, BR)])

    pltpu.make_async_copy(src_hbm.at[pl.ds(base, EPW)], srcbig, isem).wait()

    plsc.subcore_barrier()

    def dld(i, buf, sem):
        pltpu.async_copy(dst_hbm.at[pl.ds(base + i * CH, CH)], buf, sem)

    def dld_wait(i, buf, sem):
        pltpu.make_async_copy(dst_hbm.at[pl.ds(base + i * CH, CH)],
                              buf, sem).wait()

    def gat(i, buf, sem):
        pltpu.async_copy(x_hbm.at[srcbig.at[pl.ds(i * CH, CH)]], buf, sem)

    def gat_wait(i, buf, sem):
        pltpu.make_async_copy(x_hbm.at[srcbig.at[pl.ds(i * CH, CH)]],
                              buf, sem).wait()

    dld(0, dstv0, d0)
    dld(1, dstv1, d1)
    dld(2, dstv2, d2)
    gat(0, rows0, g0)
    gat(1, rows1, g1)
    gat(2, rows2, g2)

    @pl.loop(0, (NCHUNK - 2) // 3)
    def _(k):
        i = 3 * k
        for (o, rb, db, gs, ss, ds_) in ((0, rows0, dstv0, g0, s0, d0),
                                         (1, rows1, dstv1, g1, s1, d1),
                                         (2, rows2, dstv2, g2, s2, d2)):
            gat_wait(i + o, rb, gs)
            dld_wait(i + o, db, ds_)
            pltpu.async_copy(rb, acc.at[db], ss, add=True)

        for (o, rb, db, gs, ss, ds_) in ((0, rows0, dstv0, g0, s0, d0),
                                         (1, rows1, dstv1, g1, s1, d1),
                                         (2, rows2, dstv2, g2, s2, d2)):
            pltpu.make_async_copy(rb, acc.at[db], ss).wait()

            @pl.when(i + 3 + o < NCHUNK)
            def _():
                dld(i + 3 + o, db, ds_)
                gat(i + 3 + o, rb, gs)

    for (o, rb, db, gs, ss, ds_) in ((NCHUNK - 2, rows0, dstv0, g0, s0, d0),
                                     (NCHUNK - 1, rows1, dstv1, g1, s1, d1)):
        gat_wait(o, rb, gs)
        dld_wait(o, db, ds_)
        pltpu.sync_copy(rb, acc.at[db], add=True)

    plsc.subcore_barrier()

    @pl.loop(s, NB, step=NS)
    def _(bk):
        pltpu.sync_copy(acc.at[pl.ds(bk * BR, BR)], rows0)
        pltpu.sync_copy(rows0, sum_hbm.at[c, pl.ds(bk * BR, BR)])


_agg = pl.kernel(
    _agg_body,
    out_type=jax.ShapeDtypeStruct((NC, N, D), jnp.float32),
    mesh=_mesh,
    scratch_types=[
        pltpu.VMEM_SHARED((N, D), jnp.float32),
        pltpu.VMEM((CH, D), jnp.float32),
        pltpu.VMEM((CH, D), jnp.float32),
        pltpu.VMEM((CH, D), jnp.float32),
        pltpu.VMEM((EPW,), jnp.int32),
        pltpu.VMEM((CH,), jnp.int32),
        pltpu.VMEM((CH,), jnp.int32),
        pltpu.VMEM((CH,), jnp.int32),
        pltpu.SemaphoreType.DMA,
        pltpu.SemaphoreType.DMA,
        pltpu.SemaphoreType.DMA,
        pltpu.SemaphoreType.DMA,
        pltpu.SemaphoreType.DMA,
        pltpu.SemaphoreType.DMA,
        pltpu.SemaphoreType.DMA,
        pltpu.SemaphoreType.DMA,
        pltpu.SemaphoreType.DMA,
        pltpu.SemaphoreType.DMA,
    ],
)


def _cnt_body(dst_hbm, cnt_hbm, cacc, ones, dstv0, dstv1, d0, d1, s0, s1):
    c = lax.axis_index("c")
    s = lax.axis_index("s")
    wid = c * NS + s
    base = wid * EPW

    @pl.loop(0, CCH)
    def _(i):
        @pl.loop(0, D, step=16)
        def _(j):
            ones[i, pl.ds(j, 16)] = jnp.zeros((16,), jnp.float32)

    @pl.loop(s, CNB, step=NS)
    def _(bk):
        pltpu.sync_copy(ones, cacc.at[pl.ds(bk * CCH, CCH)])

    @pl.loop(0, CCH)
    def _(i):
        @pl.loop(0, D, step=16)
        def _(j):
            ones[i, pl.ds(j, 16)] = jnp.ones((16,), jnp.float32)

    plsc.subcore_barrier()

    pltpu.async_copy(dst_hbm.at[pl.ds(base, CCH)], dstv0, d0)
    pltpu.async_copy(dst_hbm.at[pl.ds(base + CCH, CCH)], dstv1, d1)

    @pl.loop(0, (CNCH - 1) // 2)
    def _(k):
        i = 2 * k
        pltpu.make_async_copy(dst_hbm.at[pl.ds(base + i * CCH, CCH)],
                              dstv0, d0).wait()
        pltpu.async_copy(ones, cacc.at[dstv0], s0, add=True)
        pltpu.make_async_copy(dst_hbm.at[pl.ds(base + (i + 1) * CCH, CCH)],
                              dstv1, d1).wait()
        pltpu.async_copy(ones, cacc.at[dstv1], s1, add=True)
        pltpu.make_async_copy(ones, cacc.at[dstv0], s0).wait()
        pltpu.async_copy(dst_hbm.at[pl.ds(base + (i + 2) * CCH, CCH)], dstv0, d0)
        pltpu.make_async_copy(ones, cacc.at[dstv1], s1).wait()

        @pl.when(i + 3 < CNCH)
        def _():
            pltpu.async_copy(dst_hbm.at[pl.ds(base + (i + 3) * CCH, CCH)],
                             dstv1, d1)

    last = CNCH - 1
    pltpu.make_async_copy(dst_hbm.at[pl.ds(base + last * CCH, CCH)],
                          dstv0, d0).wait()
    pltpu.sync_copy(ones, cacc.at[dstv0], add=True)

    plsc.subcore_barrier()

    @pl.loop(s, CNB, step=NS)
    def _(bk):
        pltpu.sync_copy(cacc.at[pl.ds(bk * CCH, CCH)], ones)
        pltpu.sync_copy(ones, cnt_hbm.at[c, pl.ds(bk * CCH, CCH)])


_cnt = pl.kernel(
    _cnt_body,
    out_type=jax.ShapeDtypeStruct((NC, N, D), jnp.float32),
    mesh=_mesh,
    scratch_types=[
        pltpu.VMEM_SHARED((N, D), jnp.float32),
        pltpu.VMEM((CCH, D), jnp.float32),
        pltpu.VMEM((CCH,), jnp.int32),
        pltpu.VMEM((CCH,), jnp.int32),
        pltpu.SemaphoreType.DMA,
        pltpu.SemaphoreType.DMA,
        pltpu.SemaphoreType.DMA,
        pltpu.SemaphoreType.DMA,
    ],
)

RB = 1000


def _combine_body(relu, p_ref, c_ref, x_ref, wl_ref, b_ref, wr_ref, o_ref):
    psum = p_ref[0] + p_ref[1]
    cnt = c_ref[0, :, 0:1] + c_ref[1, :, 0:1]
    mean = psum / jnp.maximum(cnt, 1.0)
    h = (jnp.dot(mean, wl_ref[...], preferred_element_type=jnp.float32)
         + b_ref[...]
         + jnp.dot(x_ref[...], wr_ref[...], preferred_element_type=jnp.float32))
    if relu:
        h = jnp.maximum(h, 0.0)
    o_ref[...] = h


def _combine(relu, p, cnt, x, wl_t, b, wr_t):
    return pl.pallas_call(
        functools.partial(_combine_body, relu),
        out_shape=jax.ShapeDtypeStruct((N, D), jnp.float32),
        grid=(N // RB,),
        in_specs=[
            pl.BlockSpec((NC, RB, D), lambda i: (0, i, 0)),
            pl.BlockSpec((NC, RB, D), lambda i: (0, i, 0)),
            pl.BlockSpec((RB, D), lambda i: (i, 0)),
            pl.BlockSpec((D, D), lambda i: (0, 0)),
            pl.BlockSpec((1, D), lambda i: (0, 0)),
            pl.BlockSpec((D, D), lambda i: (0, 0)),
        ],
        out_specs=pl.BlockSpec((RB, D), lambda i: (i, 0)),
    )(p, cnt, x, wl_t, b, wr_t)


def kernel(x, edge_index, W_l0, b_l0, W_r0, W_l1, b_l1, W_r1, W_l2, b_l2, W_r2):
    src = edge_index[0]
    dst = edge_index[1]

    cnt = _cnt(dst)
    p = _agg(x, src, dst)
    h = _combine(True, p, cnt, x, W_l0.T, b_l0.reshape(1, D), W_r0.T)
    p = _agg(h, src, dst)
    h = _combine(True, p, cnt, h, W_l1.T, b_l1.reshape(1, D), W_r1.T)
    p = _agg(h, src, dst)
    h = _combine(False, p, cnt, h, W_l2.T, b_l2.reshape(1, D), W_r2.T)
    return h


# speedup vs baseline: 10.8791x; 1.0396x over previous
import functools

import jax
import jax.numpy as jnp
from jax import lax
from jax.experimental import pallas as pl
from jax.experimental.pallas import tpu as pltpu
from jax.experimental.pallas import tpu_sc as plsc

N = 10000
D = 128
E = 320000
NC = 2
NS = 16
NW = NC * NS
EPW = E // NW
CH = 80
NCHUNK = EPW // CH
BR = 80
NB = N // BR
CCH = 80
CNCH = EPW // CCH
CNB = N // CCH

_mesh = plsc.VectorSubcoreMesh(core_axis_name="c", subcore_axis_name="s")


def _agg_body(x_hbm, src_hbm, dst_hbm, sum_hbm, acc,
              rows0, rows1, rows2, srcbig, dstv0, dstv1, dstv2,
              g0, g1, g2, s0, s1, s2, d0, d1, d2, isem):
    c = lax.axis_index("c")
    s = lax.axis_index("s")
    wid = c * NS + s
    base = wid * EPW

    pltpu.async_copy(src_hbm.at[pl.ds(base, EPW)], srcbig, isem)

    @pl.loop(0, CH)
    def _(i):
        @pl.loop(0, D, step=16)
        def _(j):
            rows0[i, pl.ds(j, 16)] = jnp.zeros((16,), jnp.float32)

    @pl.loop(s, NB, step=NS)
    def _(bk):
        pltpu.sync_copy(rows0, acc.at[pl.ds(bk * BR, BR)])

    pltpu.make_async_copy(src_hbm.at[pl.ds(base, EPW)], srcbig, isem).wait()

    plsc.subcore_barrier()

    def dld(i, buf, sem):
        pltpu.async_copy(dst_hbm.at[pl.ds(base + i * CH, CH)], buf, sem)

    def dld_wait(i, buf, sem):
        pltpu.make_async_copy(dst_hbm.at[pl.ds(base + i * CH, CH)],
                              buf, sem).wait()

    def gat(i, buf, sem):
        pltpu.async_copy(x_hbm.at[srcbig.at[pl.ds(i * CH, CH)]], buf, sem)

    def gat_wait(i, buf, sem):
        pltpu.make_async_copy(x_hbm.at[srcbig.at[pl.ds(i * CH, CH)]],
                              buf, sem).wait()

    dld(0, dstv0, d0)
    dld(1, dstv1, d1)
    dld(2, dstv2, d2)
    gat(0, rows0, g0)
    gat(1, rows1, g1)
    gat(2, rows2, g2)

    @pl.loop(0, (NCHUNK - 2) // 3)
    def _(k):
        i = 3 * k
        for (o, rb, db, gs, ss, ds_) in ((0, rows0, dstv0, g0, s0, d0),
                                         (1, rows1, dstv1, g1, s1, d1),
                                         (2, rows2, dstv2, g2, s2, d2)):
            gat_wait(i + o, rb, gs)
            dld_wait(i + o, db, ds_)
            pltpu.async_copy(rb, acc.at[db], ss, add=True)

        for (o, rb, db, gs, ss, ds_) in ((0, rows0, dstv0, g0, s0, d0),
                                         (1, rows1, dstv1, g1, s1, d1),
                                         (2, rows2, dstv2, g2, s2, d2)):
            pltpu.make_async_copy(rb, acc.at[db], ss).wait()

            @pl.when(i + 3 + o < NCHUNK)
            def _():
                dld(i + 3 + o, db, ds_)
                gat(i + 3 + o, rb, gs)

    for (o, rb, db, gs, ss, ds_) in ((NCHUNK - 2, rows0, dstv0, g0, s0, d0),
                                     (NCHUNK - 1, rows1, dstv1, g1, s1, d1)):
        gat_wait(o, rb, gs)
        dld_wait(o, db, ds_)
        pltpu.sync_copy(rb, acc.at[db], add=True)

    plsc.subcore_barrier()

    @pl.loop(s, NB, step=NS)
    def _(bk):
        pltpu.sync_copy(acc.at[pl.ds(bk * BR, BR)], rows0)
        pltpu.sync_copy(rows0, sum_hbm.at[c, pl.ds(bk * BR, BR)])


_agg = pl.kernel(
    _agg_body,
    out_type=jax.ShapeDtypeStruct((NC, N, D), jnp.float32),
    mesh=_mesh,
    scratch_types=[
        pltpu.VMEM_SHARED((N, D), jnp.float32),
        pltpu.VMEM((CH, D), jnp.float32),
        pltpu.VMEM((CH, D), jnp.float32),
        pltpu.VMEM((CH, D), jnp.float32),
        pltpu.VMEM((EPW,), jnp.int32),
        pltpu.VMEM((CH,), jnp.int32),
        pltpu.VMEM((CH,), jnp.int32),
        pltpu.VMEM((CH,), jnp.int32),
        pltpu.SemaphoreType.DMA,
        pltpu.SemaphoreType.DMA,
        pltpu.SemaphoreType.DMA,
        pltpu.SemaphoreType.DMA,
        pltpu.SemaphoreType.DMA,
        pltpu.SemaphoreType.DMA,
        pltpu.SemaphoreType.DMA,
        pltpu.SemaphoreType.DMA,
        pltpu.SemaphoreType.DMA,
        pltpu.SemaphoreType.DMA,
    ],
)


def _cnt_body(dst_hbm, cnt_hbm, cacc, ones,
              dstv0, dstv1, dstv2, d0, d1, d2, s0, s1, s2):
    c = lax.axis_index("c")
    s = lax.axis_index("s")
    wid = c * NS + s
    base = wid * EPW

    @pl.loop(0, CCH)
    def _(i):
        @pl.loop(0, D, step=16)
        def _(j):
            ones[i, pl.ds(j, 16)] = jnp.zeros((16,), jnp.float32)

    @pl.loop(s, CNB, step=NS)
    def _(bk):
        pltpu.sync_copy(ones, cacc.at[pl.ds(bk * CCH, CCH)])

    @pl.loop(0, CCH)
    def _(i):
        @pl.loop(0, D, step=16)
        def _(j):
            ones[i, pl.ds(j, 16)] = jnp.ones((16,), jnp.float32)

    plsc.subcore_barrier()

    def dld(i, buf, sem):
        pltpu.async_copy(dst_hbm.at[pl.ds(base + i * CCH, CCH)], buf, sem)

    def dld_wait(i, buf, sem):
        pltpu.make_async_copy(dst_hbm.at[pl.ds(base + i * CCH, CCH)],
                              buf, sem).wait()

    dld(0, dstv0, d0)
    dld(1, dstv1, d1)
    dld(2, dstv2, d2)

    @pl.loop(0, (CNCH - 2) // 3)
    def _(k):
        i = 3 * k
        for (o, db, ds_, ss) in ((0, dstv0, d0, s0),
                                 (1, dstv1, d1, s1),
                                 (2, dstv2, d2, s2)):
            dld_wait(i + o, db, ds_)
            pltpu.async_copy(ones, cacc.at[db], ss, add=True)

        for (o, db, ds_, ss) in ((0, dstv0, d0, s0),
                                 (1, dstv1, d1, s1),
                                 (2, dstv2, d2, s2)):
            pltpu.make_async_copy(ones, cacc.at[db], ss).wait()

            @pl.when(i + 3 + o < CNCH)
            def _():
                dld(i + 3 + o, db, ds_)

    for (o, db, ds_) in ((CNCH - 2, dstv0, d0), (CNCH - 1, dstv1, d1)):
        dld_wait(o, db, ds_)
        pltpu.sync_copy(ones, cacc.at[db], add=True)

    plsc.subcore_barrier()

    @pl.loop(s, CNB, step=NS)
    def _(bk):
        pltpu.sync_copy(cacc.at[pl.ds(bk * CCH, CCH)], ones)
        pltpu.sync_copy(ones, cnt_hbm.at[c, pl.ds(bk * CCH, CCH)])


_cnt = pl.kernel(
    _cnt_body,
    out_type=jax.ShapeDtypeStruct((NC, N, D), jnp.float32),
    mesh=_mesh,
    scratch_types=[
        pltpu.VMEM_SHARED((N, D), jnp.float32),
        pltpu.VMEM((CCH, D), jnp.float32),
        pltpu.VMEM((CCH,), jnp.int32),
        pltpu.VMEM((CCH,), jnp.int32),
        pltpu.VMEM((CCH,), jnp.int32),
        pltpu.SemaphoreType.DMA,
        pltpu.SemaphoreType.DMA,
        pltpu.SemaphoreType.DMA,
        pltpu.SemaphoreType.DMA,
        pltpu.SemaphoreType.DMA,
        pltpu.SemaphoreType.DMA,
    ],
)

RB = 1000


def _combine_body(relu, p_ref, c_ref, x_ref, wl_ref, b_ref, wr_ref, o_ref):
    psum = p_ref[0] + p_ref[1]
    cnt = c_ref[0, :, 0:1] + c_ref[1, :, 0:1]
    mean = psum / jnp.maximum(cnt, 1.0)
    h = (jnp.dot(mean, wl_ref[...], preferred_element_type=jnp.float32)
         + b_ref[...]
         + jnp.dot(x_ref[...], wr_ref[...], preferred_element_type=jnp.float32))
    if relu:
        h = jnp.maximum(h, 0.0)
    o_ref[...] = h


def _combine(relu, p, cnt, x, wl_t, b, wr_t):
    return pl.pallas_call(
        functools.partial(_combine_body, relu),
        out_shape=jax.ShapeDtypeStruct((N, D), jnp.float32),
        grid=(N // RB,),
        in_specs=[
            pl.BlockSpec((NC, RB, D), lambda i: (0, i, 0)),
            pl.BlockSpec((NC, RB, D), lambda i: (0, i, 0)),
            pl.BlockSpec((RB, D), lambda i: (i, 0)),
            pl.BlockSpec((D, D), lambda i: (0, 0)),
            pl.BlockSpec((1, D), lambda i: (0, 0)),
            pl.BlockSpec((D, D), lambda i: (0, 0)),
        ],
        out_specs=pl.BlockSpec((RB, D), lambda i: (i, 0)),
    )(p, cnt, x, wl_t, b, wr_t)


def kernel(x, edge_index, W_l0, b_l0, W_r0, W_l1, b_l1, W_r1, W_l2, b_l2, W_r2):
    src = edge_index[0]
    dst = edge_index[1]

    cnt = _cnt(dst)
    p = _agg(x, src, dst)
    h = _combine(True, p, cnt, x, W_l0.T, b_l0.reshape(1, D), W_r0.T)
    p = _agg(h, src, dst)
    h = _combine(True, p, cnt, h, W_l1.T, b_l1.reshape(1, D), W_r1.T)
    p = _agg(h, src, dst)
    h = _combine(False, p, cnt, h, W_l2.T, b_l2.reshape(1, D), W_r2.T)
    return h
